# baseline (speedup 1.0000x reference)
import jax
import jax.numpy as jnp
from jax.experimental import pallas as pl

_NEG_SLOPE = 0.01


def _leaky(v):
    return jnp.where(v > 0, v, v * _NEG_SLOPE)


def _transform_body(x_ref, w_ref, b_ref, o0_ref, z_ref, u_ref):
    xb = x_ref[0]
    h = jax.lax.dot_general(xb, w_ref[...], (((0,), (0,)), ((), ())),
                            preferred_element_type=jnp.float32)
    h = h + b_ref[0][None, :]
    m = h.shape[0] // 4
    h = h.reshape(m, 4, 96)
    o0_ref[0] = _leaky(h[:, :, 0:32].reshape(m, 128))
    z_ref[0] = h[:, :, 32:64].reshape(m, 128)
    u_ref[0] = h[:, :, 64:96].reshape(m, 128)


def _prop_a_body(adj_ref, z_ref, u_ref, o1_ref, pu_ref):
    a = adj_ref[0]
    o1_ref[0] = _leaky(jnp.dot(a, z_ref[0], preferred_element_type=jnp.float32))
    pu_ref[0] = jnp.dot(a, u_ref[0], preferred_element_type=jnp.float32)


def _prop_b_body(adj_ref, pu_ref, o2_ref):
    a = adj_ref[0]
    o2_ref[0] = _leaky(jnp.dot(a, pu_ref[0], preferred_element_type=jnp.float32))


def kernel(x, adj, W0, b0, W1, b1, W2, b2):
    B, F_in, N, T = x.shape
    F_out = W0.shape[1]
    C = T * F_out
    Tn = 512
    Tm = 512

    xf = x.reshape(B, F_in, N * T)
    Wall = jnp.concatenate([W0, W1, W2], axis=1)
    ball = jnp.concatenate([b0, b1, b2]).reshape(1, 3 * F_out)

    o0, z, u = pl.pallas_call(
        _transform_body,
        grid=(B, N // Tn),
        in_specs=[
            pl.BlockSpec((1, F_in, Tn * T), lambda b, i: (b, 0, i)),
            pl.BlockSpec((F_in, 3 * F_out), lambda b, i: (0, 0)),
            pl.BlockSpec((1, 3 * F_out), lambda b, i: (0, 0)),
        ],
        out_specs=[pl.BlockSpec((1, Tn, C), lambda b, i: (b, i, 0))] * 3,
        out_shape=[jax.ShapeDtypeStruct((B, N, C), jnp.float32)] * 3,
    )(xf, Wall, ball)

    o1, pu = pl.pallas_call(
        _prop_a_body,
        grid=(B, N // Tm),
        in_specs=[
            pl.BlockSpec((1, Tm, N), lambda b, i: (b, i, 0)),
            pl.BlockSpec((1, N, C), lambda b, i: (b, 0, 0)),
            pl.BlockSpec((1, N, C), lambda b, i: (b, 0, 0)),
        ],
        out_specs=[pl.BlockSpec((1, Tm, C), lambda b, i: (b, i, 0))] * 2,
        out_shape=[jax.ShapeDtypeStruct((B, N, C), jnp.float32)] * 2,
    )(adj, z, u)

    o2 = pl.pallas_call(
        _prop_b_body,
        grid=(B, N // Tm),
        in_specs=[
            pl.BlockSpec((1, Tm, N), lambda b, i: (b, i, 0)),
            pl.BlockSpec((1, N, C), lambda b, i: (b, 0, 0)),
        ],
        out_specs=pl.BlockSpec((1, Tm, C), lambda b, i: (b, i, 0)),
        out_shape=jax.ShapeDtypeStruct((B, N, C), jnp.float32),
    )(adj, pu)

    def unpack(a):
        return a.reshape(B, N, T, F_out).transpose(0, 3, 1, 2)

    return jnp.concatenate([unpack(o0), unpack(o1), unpack(o2)], axis=1)


# speedup vs baseline: 1.0000x; 1.0000x over previous
import jax
import jax.numpy as jnp
from jax.experimental import pallas as pl

_NEG_SLOPE = 0.01


def _leaky(v):
    return jnp.where(v > 0, v, v * _NEG_SLOPE)


def _transform_body(x_ref, w_ref, b_ref, o0_ref, z_ref, u_ref):
    xb = x_ref[0]
    h = jax.lax.dot_general(xb, w_ref[...], (((0,), (0,)), ((), ())),
                            preferred_element_type=jnp.float32)
    h = h + b_ref[0][None, :]
    m = h.shape[0] // 4
    h = h.reshape(m, 4, 96)
    o0_ref[0] = _leaky(h[:, :, 0:32].reshape(m, 128))
    z_ref[0] = h[:, :, 32:64].reshape(m, 128)
    u_ref[0] = h[:, :, 64:96].reshape(m, 128)


def _prop_a_body(adj_ref, z_ref, u_ref, o1_ref, pu_ref):
    a = adj_ref[0].astype(jnp.bfloat16)
    z = z_ref[0].astype(jnp.bfloat16)
    u = u_ref[0].astype(jnp.bfloat16)
    o1_ref[0] = _leaky(jnp.dot(a, z, preferred_element_type=jnp.float32))
    pu_ref[0] = jnp.dot(a, u, preferred_element_type=jnp.float32)


def _prop_b_body(adj_ref, pu_ref, o2_ref):
    a = adj_ref[0].astype(jnp.bfloat16)
    pu = pu_ref[0].astype(jnp.bfloat16)
    o2_ref[0] = _leaky(jnp.dot(a, pu, preferred_element_type=jnp.float32))


def kernel(x, adj, W0, b0, W1, b1, W2, b2):
    B, F_in, N, T = x.shape
    F_out = W0.shape[1]
    C = T * F_out
    Tn = 512
    Tm = 512

    xf = x.reshape(B, F_in, N * T)
    Wall = jnp.concatenate([W0, W1, W2], axis=1)
    ball = jnp.concatenate([b0, b1, b2]).reshape(1, 3 * F_out)

    o0, z, u = pl.pallas_call(
        _transform_body,
        grid=(B, N // Tn),
        in_specs=[
            pl.BlockSpec((1, F_in, Tn * T), lambda b, i: (b, 0, i)),
            pl.BlockSpec((F_in, 3 * F_out), lambda b, i: (0, 0)),
            pl.BlockSpec((1, 3 * F_out), lambda b, i: (0, 0)),
        ],
        out_specs=[pl.BlockSpec((1, Tn, C), lambda b, i: (b, i, 0))] * 3,
        out_shape=[jax.ShapeDtypeStruct((B, N, C), jnp.float32)] * 3,
    )(xf, Wall, ball)

    o1, pu = pl.pallas_call(
        _prop_a_body,
        grid=(B, N // Tm),
        in_specs=[
            pl.BlockSpec((1, Tm, N), lambda b, i: (b, i, 0)),
            pl.BlockSpec((1, N, C), lambda b, i: (b, 0, 0)),
            pl.BlockSpec((1, N, C), lambda b, i: (b, 0, 0)),
        ],
        out_specs=[pl.BlockSpec((1, Tm, C), lambda b, i: (b, i, 0))] * 2,
        out_shape=[jax.ShapeDtypeStruct((B, N, C), jnp.float32)] * 2,
    )(adj, z, u)

    o2 = pl.pallas_call(
        _prop_b_body,
        grid=(B, N // Tm),
        in_specs=[
            pl.BlockSpec((1, Tm, N), lambda b, i: (b, i, 0)),
            pl.BlockSpec((1, N, C), lambda b, i: (b, 0, 0)),
        ],
        out_specs=pl.BlockSpec((1, Tm, C), lambda b, i: (b, i, 0)),
        out_shape=jax.ShapeDtypeStruct((B, N, C), jnp.float32),
    )(adj, pu)

    def unpack(a):
        return a.reshape(B, N, T, F_out).transpose(0, 3, 1, 2)

    return jnp.concatenate([unpack(o0), unpack(o1), unpack(o2)], axis=1)


# speedup vs baseline: 1.0313x; 1.0312x over previous
import jax
import jax.numpy as jnp
from jax.experimental import pallas as pl
from jax.experimental.pallas import tpu as pltpu

_NEG_SLOPE = 0.01


def _leaky(v):
    return jnp.where(v > 0, v, v * _NEG_SLOPE)


def _mixhop_body(x_ref, adj_ref, w_ref, b_ref, o_ref, z_ref, u_ref, pu_ref):
    ph = pl.program_id(1)
    i = pl.program_id(2)
    tm = adj_ref.shape[1]

    @pl.when(ph == 0)
    def _transform():
        xb = x_ref[0]
        h = jax.lax.dot_general(xb, w_ref[...], (((0,), (0,)), ((), ())),
                                preferred_element_type=jnp.float32)
        h = h + b_ref[0][None, :]
        h = h.reshape(tm, 4, 96)
        o_ref[0, 0] = _leaky(h[:, :, 0:32].reshape(tm, 128))
        z_ref[pl.ds(i * tm, tm), :] = h[:, :, 32:64].reshape(tm, 128)
        u_ref[pl.ds(i * tm, tm), :] = h[:, :, 64:96].reshape(tm, 128)

    @pl.when(ph == 1)
    def _hop1():
        a = adj_ref[0].astype(jnp.bfloat16)
        z = z_ref[...].astype(jnp.bfloat16)
        u = u_ref[...].astype(jnp.bfloat16)
        o_ref[0, 0] = _leaky(jnp.dot(a, z, preferred_element_type=jnp.float32))
        pu_ref[pl.ds(i * tm, tm), :] = jnp.dot(
            a, u, preferred_element_type=jnp.float32)

    @pl.when(ph == 2)
    def _hop2():
        a = adj_ref[0].astype(jnp.bfloat16)
        pu = pu_ref[...].astype(jnp.bfloat16)
        o_ref[0, 0] = _leaky(jnp.dot(a, pu, preferred_element_type=jnp.float32))


def kernel(x, adj, W0, b0, W1, b1, W2, b2):
    B, F_in, N, T = x.shape
    F_out = W0.shape[1]
    C = T * F_out
    Tm = 512

    xf = x.reshape(B, F_in, N * T)
    Wall = jnp.concatenate([W0, W1, W2], axis=1)
    ball = jnp.concatenate([b0, b1, b2]).reshape(1, 3 * F_out)

    stacked = pl.pallas_call(
        _mixhop_body,
        grid=(B, 3, N // Tm),
        in_specs=[
            pl.BlockSpec((1, F_in, Tm * T),
                         lambda b, ph, i: (b, 0, jnp.where(ph == 0, i, 0))),
            pl.BlockSpec((1, Tm, N),
                         lambda b, ph, i: (b, jnp.where(ph == 0, 0, i), 0)),
            pl.BlockSpec((F_in, 3 * F_out), lambda b, ph, i: (0, 0)),
            pl.BlockSpec((1, 3 * F_out), lambda b, ph, i: (0, 0)),
        ],
        out_specs=pl.BlockSpec((1, 1, Tm, C), lambda b, ph, i: (b, ph, i, 0)),
        out_shape=jax.ShapeDtypeStruct((B, 3, N, C), jnp.float32),
        scratch_shapes=[
            pltpu.VMEM((N, C), jnp.float32),
            pltpu.VMEM((N, C), jnp.float32),
            pltpu.VMEM((N, C), jnp.float32),
        ],
    )(xf, adj, Wall, ball)

    out = stacked.reshape(B, 3, N, T, F_out).transpose(0, 1, 4, 2, 3)
    return out.reshape(B, 3 * F_out, N, T)
